# baseline (speedup 1.0000x reference)
import functools

import jax
import jax.numpy as jnp
from jax.experimental import pallas as pl
from jax.experimental.pallas import tpu as pltpu

BM = 1024
BN = 512


NCHUNK = 4


def _linear_kernel(x_hbm, w_ref, b_ref, o_ref, xbuf_ref, sems):
    i = pl.program_id(0)
    j = pl.program_id(1)
    ni = pl.num_programs(0)
    ch = BM // NCHUNK

    def _chunk_copy(strip, c):
        slot = jax.lax.rem(strip, 2)
        return pltpu.make_async_copy(
            x_hbm.at[pl.ds(strip * BM + c * ch, ch)],
            xbuf_ref.at[slot, pl.ds(c * ch, ch)],
            sems.at[slot, c],
        )

    @pl.when(jnp.logical_and(i == 0, j == 0))
    def _start_first_strip():
        for c in range(NCHUNK):
            _chunk_copy(0, c).start()

    @pl.when(j == 0)
    def _wait_strip():
        for c in range(NCHUNK):
            _chunk_copy(i, c).wait()

    @pl.when(jnp.logical_and(i + 1 < ni, jnp.logical_and(1 <= j, j <= NCHUNK)))
    def _start_next_chunk():
        _chunk_copy(i + 1, j - 1).start()

    x = xbuf_ref[i % 2].astype(jnp.bfloat16)
    w = w_ref[...].astype(jnp.bfloat16)
    acc = jnp.dot(x, w, preferred_element_type=jnp.float32)
    o_ref[...] = acc + b_ref[...]


@functools.partial(jax.jit, static_argnames=())
def kernel(input, W, b):
    m, k = input.shape
    _, n = W.shape
    b2 = b.reshape(1, n)
    W = W.astype(jnp.bfloat16)
    grid = (m // BM, n // BN)
    return pl.pallas_call(
        _linear_kernel,
        grid=grid,
        in_specs=[
            pl.BlockSpec(memory_space=pl.ANY),
            pl.BlockSpec((k, BN), lambda i, j: (0, j)),
            pl.BlockSpec((1, BN), lambda i, j: (0, j)),
        ],
        out_specs=pl.BlockSpec((BM, BN), lambda i, j: (i, j)),
        out_shape=jax.ShapeDtypeStruct((m, n), jnp.float32),
        scratch_shapes=[
            pltpu.VMEM((2, BM, k), jnp.float32),
            pltpu.SemaphoreType.DMA((2, NCHUNK)),
        ],
        compiler_params=pltpu.CompilerParams(
            dimension_semantics=("arbitrary", "arbitrary"),
        ),
    )(input, W, b2)


# speedup vs baseline: 1.0836x; 1.0836x over previous
import functools

import jax
import jax.numpy as jnp
from jax.experimental import pallas as pl
from jax.experimental.pallas import tpu as pltpu

BM = 1024
BN = 512


NCHUNK = 4


def _linear_kernel(x_hbm, w_ref, b_ref, o_ref, xbuf_ref, sems):
    i = pl.program_id(0)
    j = pl.program_id(1)
    ni = pl.num_programs(0)
    ch = BM // NCHUNK

    def _chunk_copy(strip, c):
        slot = jax.lax.rem(strip, 2)
        return pltpu.make_async_copy(
            x_hbm.at[pl.ds(strip * BM + c * ch, ch)],
            xbuf_ref.at[slot, pl.ds(c * ch, ch)],
            sems.at[slot, c],
        )

    @pl.when(jnp.logical_and(i == 0, j == 0))
    def _start_first_strip():
        for c in range(NCHUNK):
            _chunk_copy(0, c).start()

    @pl.when(j == 0)
    def _wait_strip():
        for c in range(NCHUNK):
            _chunk_copy(i, c).wait()

    @pl.when(jnp.logical_and(i + 1 < ni, jnp.logical_and(1 <= j, j <= NCHUNK)))
    def _start_next_chunk():
        _chunk_copy(i + 1, j - 1).start()

    x = xbuf_ref[i % 2].astype(jnp.bfloat16)
    w = w_ref[...].astype(jnp.bfloat16)
    acc = jnp.dot(x, w, preferred_element_type=jnp.float32)
    o_ref[...] = acc + b_ref[...]


@functools.partial(jax.jit, static_argnames=())
def kernel(input, W, b):
    m, k = input.shape
    _, n = W.shape
    b2 = b.reshape(1, n)
    grid = (m // BM, n // BN)
    nj = n // BN

    def _snake(i, j):
        return jnp.where(i % 2 == 0, j, nj - 1 - j)
    return pl.pallas_call(
        _linear_kernel,
        grid=grid,
        in_specs=[
            pl.BlockSpec(memory_space=pl.ANY),
            pl.BlockSpec((k, BN), lambda i, j: (0, _snake(i, j))),
            pl.BlockSpec((1, BN), lambda i, j: (0, _snake(i, j))),
        ],
        out_specs=pl.BlockSpec((BM, BN), lambda i, j: (i, _snake(i, j))),
        out_shape=jax.ShapeDtypeStruct((m, n), jnp.float32),
        scratch_shapes=[
            pltpu.VMEM((2, BM, k), jnp.float32),
            pltpu.SemaphoreType.DMA((2, NCHUNK)),
        ],
        compiler_params=pltpu.CompilerParams(
            dimension_semantics=("arbitrary", "arbitrary"),
        ),
    )(input, W, b2)


# speedup vs baseline: 1.0904x; 1.0062x over previous
import functools

import jax
import jax.numpy as jnp
from jax.experimental import pallas as pl
from jax.experimental.pallas import tpu as pltpu

BM = 1024
BN = 512


NCHUNK = 4


def _linear_kernel(x_hbm, w_ref, b_ref, o_ref, xbuf_ref, sems):
    i = pl.program_id(0)
    j = pl.program_id(1)
    ni = pl.num_programs(0)
    ch = BM // NCHUNK

    def _chunk_copy(strip, c):
        slot = jax.lax.rem(strip, 2)
        return pltpu.make_async_copy(
            x_hbm.at[pl.ds(strip * BM + c * ch, ch)],
            xbuf_ref.at[slot, pl.ds(c * ch, ch)],
            sems.at[slot, c],
        )

    @pl.when(jnp.logical_and(i == 0, j == 0))
    def _start_first_strip():
        for c in range(NCHUNK):
            _chunk_copy(0, c).start()

    @pl.when(j == 0)
    def _wait_strip():
        for c in range(NCHUNK):
            _chunk_copy(i, c).wait()

    @pl.when(jnp.logical_and(i + 1 < ni, jnp.logical_and(1 <= j, j <= NCHUNK)))
    def _start_next_chunk():
        _chunk_copy(i + 1, j - 1).start()

    x = xbuf_ref[i % 2].astype(jnp.bfloat16)
    w = w_ref[...].astype(jnp.bfloat16)
    h = BN // 2
    acc0 = jnp.dot(x, w[:, :h], preferred_element_type=jnp.float32)
    o_ref[:, :h] = acc0 + b_ref[:, :h]
    acc1 = jnp.dot(x, w[:, h:], preferred_element_type=jnp.float32)
    o_ref[:, h:] = acc1 + b_ref[:, h:]


@functools.partial(jax.jit, static_argnames=())
def kernel(input, W, b):
    m, k = input.shape
    _, n = W.shape
    b2 = b.reshape(1, n)
    grid = (m // BM, n // BN)
    return pl.pallas_call(
        _linear_kernel,
        grid=grid,
        in_specs=[
            pl.BlockSpec(memory_space=pl.ANY),
            pl.BlockSpec((k, BN), lambda i, j: (0, j)),
            pl.BlockSpec((1, BN), lambda i, j: (0, j)),
        ],
        out_specs=pl.BlockSpec((BM, BN), lambda i, j: (i, j)),
        out_shape=jax.ShapeDtypeStruct((m, n), jnp.float32),
        scratch_shapes=[
            pltpu.VMEM((2, BM, k), jnp.float32),
            pltpu.SemaphoreType.DMA((2, NCHUNK)),
        ],
        compiler_params=pltpu.CompilerParams(
            dimension_semantics=("arbitrary", "arbitrary"),
        ),
    )(input, W, b2)


# speedup vs baseline: 1.0984x; 1.0073x over previous
import functools

import jax
import jax.numpy as jnp
from jax.experimental import pallas as pl
from jax.experimental.pallas import tpu as pltpu

BM = 1024
BN = 512


NCHUNK = 4


def _linear_kernel(x_hbm, w_ref, b_ref, o_ref, xbuf_ref, sems):
    i = pl.program_id(0)
    j = pl.program_id(1)
    ni = pl.num_programs(0)
    ch = BM // NCHUNK

    def _chunk_copy(strip, c):
        slot = jax.lax.rem(strip, 2)
        return pltpu.make_async_copy(
            x_hbm.at[pl.ds(strip * BM + c * ch, ch)],
            xbuf_ref.at[slot, pl.ds(c * ch, ch)],
            sems.at[slot, c],
        )

    @pl.when(jnp.logical_and(i == 0, j == 0))
    def _start_first_strip():
        for c in range(NCHUNK):
            _chunk_copy(0, c).start()

    @pl.when(j == 0)
    def _wait_strip():
        for c in range(NCHUNK):
            _chunk_copy(i, c).wait()

    @pl.when(jnp.logical_and(i + 1 < ni, jnp.logical_and(1 <= j, j <= NCHUNK)))
    def _start_next_chunk():
        _chunk_copy(i + 1, j - 1).start()

    x = xbuf_ref[i % 2].astype(jnp.bfloat16)
    w = w_ref[...].astype(jnp.bfloat16)
    h = BN // 2
    r = BM // 2
    for ri in range(2):
        for ci in range(2):
            acc = jnp.dot(
                x[ri * r:(ri + 1) * r, :],
                w[:, ci * h:(ci + 1) * h],
                preferred_element_type=jnp.float32,
            )
            o_ref[ri * r:(ri + 1) * r, ci * h:(ci + 1) * h] = (
                acc + b_ref[:, ci * h:(ci + 1) * h]
            )


@functools.partial(jax.jit, static_argnames=())
def kernel(input, W, b):
    m, k = input.shape
    _, n = W.shape
    b2 = b.reshape(1, n)
    grid = (m // BM, n // BN)
    return pl.pallas_call(
        _linear_kernel,
        grid=grid,
        in_specs=[
            pl.BlockSpec(memory_space=pl.ANY),
            pl.BlockSpec((k, BN), lambda i, j: (0, j)),
            pl.BlockSpec((1, BN), lambda i, j: (0, j)),
        ],
        out_specs=pl.BlockSpec((BM, BN), lambda i, j: (i, j)),
        out_shape=jax.ShapeDtypeStruct((m, n), jnp.float32),
        scratch_shapes=[
            pltpu.VMEM((2, BM, k), jnp.float32),
            pltpu.SemaphoreType.DMA((2, NCHUNK)),
        ],
        compiler_params=pltpu.CompilerParams(
            dimension_semantics=("arbitrary", "arbitrary"),
        ),
    )(input, W, b2)
